# baseline (speedup 1.0000x reference)
import functools

import jax
import jax.numpy as jnp
from jax import lax
from jax.experimental import pallas as pl
from jax.experimental.pallas import tpu as pltpu
from jax.experimental.pallas import tpu_sc as plsc

N_NODES = 10000
N_PAD = 10240
D = 128
E_PAD = 32 * 80 * 128
NW = 32
EDGES_PER_TILE = E_PAD // NW
CHUNKS = 160
CHUNK = 64
ROWS_PER_TILE = N_PAD // 16

_mesh = plsc.VectorSubcoreMesh(core_axis_name="c", subcore_axis_name="s")


def _deg_body(dst_hbm, deg_hbm, part_hbm, dstv, histv, rbuf, sbuf):
    c = lax.axis_index("c")
    s = lax.axis_index("s")
    w = c * 16 + s
    pltpu.sync_copy(dst_hbm.at[pl.ds(w * EDGES_PER_TILE, EDGES_PER_TILE)], dstv)
    zeros = jnp.zeros((16,), jnp.float32)
    ones = jnp.full((16,), 1.0, jnp.float32)

    def zbody(i, carry):
        histv[pl.ds(i * 16, 16)] = zeros
        return carry

    lax.fori_loop(0, N_PAD // 16, zbody, 0)

    def body(i, carry):
        idx = dstv[pl.ds(i * 16, 16)]
        plsc.addupdate_scatter(histv, [idx], ones)
        return carry

    lax.fori_loop(0, EDGES_PER_TILE // 16, body, 0)
    pltpu.sync_copy(histv, part_hbm.at[c, s])
    plsc.subcore_barrier()
    pltpu.sync_copy(part_hbm.at[c, :, pl.ds(s * 640, 640)], rbuf)

    def rbody(i, carry):
        sl = pl.ds(i * 16, 16)
        v = rbuf[0, sl]
        for t in range(1, 16):
            v = v + rbuf[t, sl]
        sbuf[sl] = v
        return carry

    lax.fori_loop(0, 640 // 16, rbody, 0)
    pltpu.sync_copy(sbuf, deg_hbm.at[c, pl.ds(s * 640, 640)])


_deg_call = functools.partial(
    pl.kernel,
    out_type=(
        jax.ShapeDtypeStruct((2, N_PAD), jnp.float32),
        jax.ShapeDtypeStruct((2, 16, N_PAD), jnp.float32),
    ),
    mesh=_mesh,
    compiler_params=pltpu.CompilerParams(needs_layout_passes=False),
    scratch_types=[
        pltpu.VMEM((EDGES_PER_TILE,), jnp.int32),
        pltpu.VMEM((N_PAD,), jnp.float32),
        pltpu.VMEM((16, 640), jnp.float32),
        pltpu.VMEM((640,), jnp.float32),
    ],
)(_deg_body)


def _mm_body(x_ref, w_ref, deg_ref, h2_ref, h2b_ref):
    deg = deg_ref[0] + deg_ref[1] + 1.0
    dis = lax.rsqrt(deg)[:, None]
    h = jnp.dot(x_ref[...], w_ref[...], preferred_element_type=jnp.float32)
    h2 = h * dis
    h2_ref[...] = h2
    ub = lax.bitcast_convert_type(h2, jnp.int32)
    ub = ub + jnp.int32(0x8000)
    groups = []
    for g in range(4):
        a = ub[:, g * 32:g * 32 + 16]
        bb = ub[:, g * 32 + 16:g * 32 + 32]
        lo = lax.shift_right_logical(a, 16) & jnp.int32(0xFFFF)
        hi = bb & jnp.int32(-65536)
        groups.append(lo | hi)
    h2b_ref[...] = jnp.concatenate(groups, axis=1)


def _mm_call(x_p, W, deg2):
    return pl.pallas_call(
        _mm_body,
        grid=(N_PAD // 1024,),
        in_specs=[
            pl.BlockSpec((1024, D), lambda i: (i, 0)),
            pl.BlockSpec((D, D), lambda i: (0, 0)),
            pl.BlockSpec((2, 1024), lambda i: (0, i)),
        ],
        out_specs=[
            pl.BlockSpec((1024, D), lambda i: (i, 0)),
            pl.BlockSpec((1024, D // 2), lambda i: (i, 0)),
        ],
        out_shape=[
            jax.ShapeDtypeStruct((N_PAD, D), jnp.float32),
            jax.ShapeDtypeStruct((N_PAD, D // 2), jnp.int32),
        ],
    )(x_p, W, deg2)


CH_F = 306
CH_S = (E_PAD // CHUNK) // 16 - CH_F


def _agg_body(h2_hbm, h2b_hbm, edge_hbm, part_hbm,
              idxv, gbuf, sbuf, acc, isem, gsem, ssem):
    c = lax.axis_index("c")
    s = lax.axis_index("s")
    nch = jnp.where(c == 0, CH_F, CH_S)
    base = jnp.where(c == 0, s * CH_F, 16 * CH_F + s * CH_S)
    pltpu.sync_copy(h2_hbm.at[pl.ds(s * ROWS_PER_TILE, ROWS_PER_TILE)],
                    acc.at[pl.ds(s * ROWS_PER_TILE, ROWS_PER_TILE)])
    plsc.subcore_barrier()

    for p in range(12):
        pltpu.async_copy(edge_hbm.at[base + p], idxv.at[p], isem)
    for p in range(4):
        pltpu.make_async_copy(edge_hbm.at[base + p], idxv.at[p], isem).wait()
        pltpu.async_copy(h2b_hbm.at[idxv.at[p, 0]], gbuf.at[p], gsem)

    def body(j, carry):
        b = lax.rem(j, 4)
        d = lax.rem(j, 2)
        sl = lax.rem(j, 16)
        pltpu.make_async_copy(h2b_hbm.at[idxv.at[sl, 0]], gbuf.at[b], gsem).wait()

        gb = gbuf.at[b]
        sb = sbuf.at[d]

        def conv(r4, carry2):
            for rr in range(4):
                r = r4 * 4 + rr
                for g in range(4):
                    wi = gb[r, pl.ds(g * 16, 16)]
                    lo = plsc.bitcast(wi << 16, jnp.float32)
                    hi = plsc.bitcast(wi & jnp.int32(-65536), jnp.float32)
                    sb[r, pl.ds(g * 32, 16)] = lo
                    sb[r, pl.ds(g * 32 + 16, 16)] = hi
            return carry2

        lax.fori_loop(0, CHUNK // 4, conv, 0)

        @pl.when(j >= 2)
        def _():
            sld = lax.rem(j - 2, 16)
            pltpu.make_async_copy(sbuf.at[d], acc.at[idxv.at[sld, 1]], ssem).wait()

        pltpu.async_copy(sbuf.at[d], acc.at[idxv.at[sl, 1]], ssem, add=True)

        @pl.when(j + 4 < nch)
        def _():
            sl4 = lax.rem(j + 4, 16)
            pltpu.make_async_copy(edge_hbm.at[base + j + 4], idxv.at[sl4], isem).wait()
            pltpu.async_copy(h2b_hbm.at[idxv.at[sl4, 0]], gbuf.at[b], gsem)

        @pl.when(j + 12 < nch)
        def _():
            sl12 = lax.rem(j + 12, 16)
            pltpu.async_copy(edge_hbm.at[base + j + 12], idxv.at[sl12], isem)

        return carry

    lax.fori_loop(0, nch, body, 0)
    pltpu.make_async_copy(sbuf.at[lax.rem(nch - 2, 2)],
                          acc.at[idxv.at[lax.rem(nch - 2, 16), 1]], ssem).wait()
    pltpu.make_async_copy(sbuf.at[lax.rem(nch - 1, 2)],
                          acc.at[idxv.at[lax.rem(nch - 1, 16), 1]], ssem).wait()
    plsc.subcore_barrier()
    pltpu.sync_copy(acc.at[pl.ds(s * ROWS_PER_TILE, ROWS_PER_TILE)],
                    part_hbm.at[c, pl.ds(s * ROWS_PER_TILE, ROWS_PER_TILE)])


_agg_call = functools.partial(
    pl.kernel,
    out_type=jax.ShapeDtypeStruct((2, N_PAD, D), jnp.float32),
    mesh=_mesh,
    compiler_params=pltpu.CompilerParams(needs_layout_passes=False,
                                         use_tc_tiling_on_sc=False),
    scratch_types=[
        pltpu.VMEM((16, 2, CHUNK), jnp.int32),
        pltpu.VMEM((4, CHUNK, D // 2), jnp.int32),
        pltpu.VMEM((2, CHUNK, D), jnp.float32),
        pltpu.VMEM_SHARED((N_PAD, D), jnp.float32),
        pltpu.SemaphoreType.DMA,
        pltpu.SemaphoreType.DMA,
        pltpu.SemaphoreType.DMA,
    ],
)(_agg_body)


def _fin_body(part_ref, h2_ref, deg_ref, b_ref, out_ref):
    deg = deg_ref[0] + deg_ref[1] + 1.0
    dis = lax.rsqrt(deg)[:, None]
    agg = part_ref[0] + part_ref[1] - h2_ref[...]
    out_ref[...] = agg * dis + b_ref[...]


def _fin_call(part, h2, deg2, b2):
    return pl.pallas_call(
        _fin_body,
        grid=(N_PAD // 1024,),
        in_specs=[
            pl.BlockSpec((2, 1024, D), lambda i: (0, i, 0)),
            pl.BlockSpec((1024, D), lambda i: (i, 0)),
            pl.BlockSpec((2, 1024), lambda i: (0, i)),
            pl.BlockSpec((1, D), lambda i: (0, 0)),
        ],
        out_specs=pl.BlockSpec((1024, D), lambda i: (i, 0)),
        out_shape=jax.ShapeDtypeStruct((N_NODES, D), jnp.float32),
    )(part, h2, deg2, b2)


def kernel(x, edge_index, W, b):
    src = edge_index[0].astype(jnp.int32)
    dst = edge_index[1].astype(jnp.int32)
    pad = E_PAD - src.shape[0]
    src_p = jnp.concatenate([src, jnp.zeros((pad,), jnp.int32)])
    dst_p = jnp.concatenate([dst, jnp.full((pad,), N_NODES, jnp.int32)])
    edge3 = jnp.stack(
        [src_p.reshape(E_PAD // CHUNK, CHUNK), dst_p.reshape(E_PAD // CHUNK, CHUNK)],
        axis=1)
    x_p = jnp.concatenate([x, jnp.zeros((N_PAD - N_NODES, D), x.dtype)])

    deg2, _ = _deg_call(dst_p)
    h2, h2b = _mm_call(x_p, W, deg2)
    part = _agg_call(h2, h2b, edge3)
    return _fin_call(part, h2, deg2, b.reshape(1, D))


# speedup vs baseline: 1.1345x; 1.1345x over previous
import functools

import jax
import jax.numpy as jnp
from jax import lax
from jax.experimental import pallas as pl
from jax.experimental.pallas import tpu as pltpu
from jax.experimental.pallas import tpu_sc as plsc

N_NODES = 10000
N_PAD = 10240
D = 128
E_PAD = 32 * 80 * 128
NW = 32
EDGES_PER_TILE = E_PAD // NW
CHUNKS = 160
CHUNK = 64
ROWS_PER_TILE = N_PAD // 16

_mesh = plsc.VectorSubcoreMesh(core_axis_name="c", subcore_axis_name="s")


def _deg_body(dst_hbm, deg_hbm, part_hbm, dstv, histv, rbuf, sbuf):
    c = lax.axis_index("c")
    s = lax.axis_index("s")
    w = c * 16 + s
    pltpu.sync_copy(dst_hbm.at[pl.ds(w * EDGES_PER_TILE, EDGES_PER_TILE)], dstv)
    zeros = jnp.zeros((16,), jnp.float32)
    ones = jnp.full((16,), 1.0, jnp.float32)

    def zbody(i, carry):
        histv[pl.ds(i * 16, 16)] = zeros
        return carry

    lax.fori_loop(0, N_PAD // 16, zbody, 0)

    def body(i, carry):
        idx = dstv[pl.ds(i * 16, 16)]
        plsc.addupdate_scatter(histv, [idx], ones)
        return carry

    lax.fori_loop(0, EDGES_PER_TILE // 16, body, 0)
    pltpu.sync_copy(histv, part_hbm.at[c, s])
    plsc.subcore_barrier()
    pltpu.sync_copy(part_hbm.at[c, :, pl.ds(s * 640, 640)], rbuf)

    def rbody(i, carry):
        sl = pl.ds(i * 16, 16)
        v = rbuf[0, sl]
        for t in range(1, 16):
            v = v + rbuf[t, sl]
        sbuf[sl] = v
        return carry

    lax.fori_loop(0, 640 // 16, rbody, 0)
    pltpu.sync_copy(sbuf, deg_hbm.at[c, pl.ds(s * 640, 640)])


_deg_call = functools.partial(
    pl.kernel,
    out_type=(
        jax.ShapeDtypeStruct((2, N_PAD), jnp.float32),
        jax.ShapeDtypeStruct((2, 16, N_PAD), jnp.float32),
    ),
    mesh=_mesh,
    compiler_params=pltpu.CompilerParams(needs_layout_passes=False),
    scratch_types=[
        pltpu.VMEM((EDGES_PER_TILE,), jnp.int32),
        pltpu.VMEM((N_PAD,), jnp.float32),
        pltpu.VMEM((16, 640), jnp.float32),
        pltpu.VMEM((640,), jnp.float32),
    ],
)(_deg_body)


def _mm_body(x_ref, w_ref, deg_ref, h2_ref):
    deg = deg_ref[0] + deg_ref[1] + 1.0
    dis = lax.rsqrt(deg)[:, None]
    h = jnp.dot(x_ref[...], w_ref[...], preferred_element_type=jnp.float32)
    h2_ref[...] = h * dis


def _mm_call(x_p, W, deg2):
    return pl.pallas_call(
        _mm_body,
        grid=(N_PAD // 1024,),
        in_specs=[
            pl.BlockSpec((1024, D), lambda i: (i, 0)),
            pl.BlockSpec((D, D), lambda i: (0, 0)),
            pl.BlockSpec((2, 1024), lambda i: (0, i)),
        ],
        out_specs=pl.BlockSpec((1024, D), lambda i: (i, 0)),
        out_shape=jax.ShapeDtypeStruct((N_PAD, D), jnp.float32),
    )(x_p, W, deg2)


CH_F = 296
CH_S = (E_PAD // CHUNK) // 16 - CH_F


def _agg_body(h2_hbm, edge_hbm, part_hbm, idxv, gbuf, acc, isem, gsem):
    c = lax.axis_index("c")
    s = lax.axis_index("s")
    nch = jnp.where(c == 0, CH_F, CH_S)
    base = jnp.where(c == 0, s * CH_F, 16 * CH_F + s * CH_S)
    pltpu.sync_copy(h2_hbm.at[pl.ds(s * ROWS_PER_TILE, ROWS_PER_TILE)],
                    acc.at[pl.ds(s * ROWS_PER_TILE, ROWS_PER_TILE)])
    plsc.subcore_barrier()

    for p in range(12):
        pltpu.async_copy(edge_hbm.at[base + p], idxv.at[p], isem)
    for p in range(4):
        pltpu.make_async_copy(edge_hbm.at[base + p], idxv.at[p], isem).wait()
        pltpu.async_copy(h2_hbm.at[idxv.at[p, 0]], gbuf.at[p], gsem)

    def body(j, carry):
        b = lax.rem(j, 4)
        sl = lax.rem(j, 16)
        pltpu.make_async_copy(h2_hbm.at[idxv.at[sl, 0]], gbuf.at[b], gsem).wait()
        pltpu.sync_copy(gbuf.at[b], acc.at[idxv.at[sl, 1]], add=True)

        @pl.when(j + 4 < nch)
        def _():
            sl4 = lax.rem(j + 4, 16)
            pltpu.make_async_copy(edge_hbm.at[base + j + 4], idxv.at[sl4], isem).wait()
            pltpu.async_copy(h2_hbm.at[idxv.at[sl4, 0]], gbuf.at[b], gsem)

        @pl.when(j + 12 < nch)
        def _():
            sl12 = lax.rem(j + 12, 16)
            pltpu.async_copy(edge_hbm.at[base + j + 12], idxv.at[sl12], isem)

        return carry

    lax.fori_loop(0, nch, body, 0)
    plsc.subcore_barrier()
    pltpu.sync_copy(acc.at[pl.ds(s * ROWS_PER_TILE, ROWS_PER_TILE)],
                    part_hbm.at[c, pl.ds(s * ROWS_PER_TILE, ROWS_PER_TILE)])


_agg_call = functools.partial(
    pl.kernel,
    out_type=jax.ShapeDtypeStruct((2, N_PAD, D), jnp.float32),
    mesh=_mesh,
    scratch_types=[
        pltpu.VMEM((16, 2, CHUNK), jnp.int32),
        pltpu.VMEM((4, CHUNK, D), jnp.float32),
        pltpu.VMEM_SHARED((N_PAD, D), jnp.float32),
        pltpu.SemaphoreType.DMA,
        pltpu.SemaphoreType.DMA,
    ],
)(_agg_body)


def _fin_body(part_ref, h2_ref, deg_ref, b_ref, out_ref):
    deg = deg_ref[0] + deg_ref[1] + 1.0
    dis = lax.rsqrt(deg)[:, None]
    agg = part_ref[0] + part_ref[1] - h2_ref[...]
    out_ref[...] = agg * dis + b_ref[...]


def _fin_call(part, h2, deg2, b2):
    return pl.pallas_call(
        _fin_body,
        grid=(N_PAD // 1024,),
        in_specs=[
            pl.BlockSpec((2, 1024, D), lambda i: (0, i, 0)),
            pl.BlockSpec((1024, D), lambda i: (i, 0)),
            pl.BlockSpec((2, 1024), lambda i: (0, i)),
            pl.BlockSpec((1, D), lambda i: (0, 0)),
        ],
        out_specs=pl.BlockSpec((1024, D), lambda i: (i, 0)),
        out_shape=jax.ShapeDtypeStruct((N_NODES, D), jnp.float32),
    )(part, h2, deg2, b2)


def kernel(x, edge_index, W, b):
    src = edge_index[0].astype(jnp.int32)
    dst = edge_index[1].astype(jnp.int32)
    pad = E_PAD - src.shape[0]
    src_p = jnp.concatenate([src, jnp.zeros((pad,), jnp.int32)])
    dst_p = jnp.concatenate([dst, jnp.full((pad,), N_NODES, jnp.int32)])
    edge3 = jnp.stack(
        [src_p.reshape(E_PAD // CHUNK, CHUNK), dst_p.reshape(E_PAD // CHUNK, CHUNK)],
        axis=1)
    x_p = jnp.concatenate([x, jnp.zeros((N_PAD - N_NODES, D), x.dtype)])

    deg2, _ = _deg_call(dst_p)
    h2 = _mm_call(x_p, W, deg2)
    part = _agg_call(h2, edge3)
    return _fin_call(part, h2, deg2, b.reshape(1, D))


# speedup vs baseline: 1.1382x; 1.0032x over previous
import functools

import jax
import jax.numpy as jnp
from jax import lax
from jax.experimental import pallas as pl
from jax.experimental.pallas import tpu as pltpu
from jax.experimental.pallas import tpu_sc as plsc

N_NODES = 10000
N_PAD = 10240
D = 128
E_PAD = 32 * 80 * 128
NW = 32
EDGES_PER_TILE = E_PAD // NW
CHUNKS = 160
CHUNK = 64
ROWS_PER_TILE = N_PAD // 16

_mesh = plsc.VectorSubcoreMesh(core_axis_name="c", subcore_axis_name="s")


def _deg_body(dst_hbm, deg_hbm, part_hbm, dstv, histv, rbuf, sbuf):
    c = lax.axis_index("c")
    s = lax.axis_index("s")
    w = c * 16 + s
    pltpu.sync_copy(dst_hbm.at[pl.ds(w * EDGES_PER_TILE, EDGES_PER_TILE)], dstv)
    zeros = jnp.zeros((16,), jnp.float32)
    ones = jnp.full((16,), 1.0, jnp.float32)

    def zbody(i, carry):
        histv[pl.ds(i * 16, 16)] = zeros
        return carry

    lax.fori_loop(0, N_PAD // 16, zbody, 0)

    def body(i, carry):
        idx = dstv[pl.ds(i * 16, 16)]
        plsc.addupdate_scatter(histv, [idx], ones)
        return carry

    lax.fori_loop(0, EDGES_PER_TILE // 16, body, 0)
    pltpu.sync_copy(histv, part_hbm.at[c, s])
    plsc.subcore_barrier()
    pltpu.sync_copy(part_hbm.at[c, :, pl.ds(s * 640, 640)], rbuf)

    def rbody(i, carry):
        sl = pl.ds(i * 16, 16)
        v = rbuf[0, sl]
        for t in range(1, 16):
            v = v + rbuf[t, sl]
        sbuf[sl] = v
        return carry

    lax.fori_loop(0, 640 // 16, rbody, 0)
    pltpu.sync_copy(sbuf, deg_hbm.at[c, pl.ds(s * 640, 640)])


_deg_call = functools.partial(
    pl.kernel,
    out_type=(
        jax.ShapeDtypeStruct((2, N_PAD), jnp.float32),
        jax.ShapeDtypeStruct((2, 16, N_PAD), jnp.float32),
    ),
    mesh=_mesh,
    compiler_params=pltpu.CompilerParams(needs_layout_passes=False),
    scratch_types=[
        pltpu.VMEM((EDGES_PER_TILE,), jnp.int32),
        pltpu.VMEM((N_PAD,), jnp.float32),
        pltpu.VMEM((16, 640), jnp.float32),
        pltpu.VMEM((640,), jnp.float32),
    ],
)(_deg_body)


def _mm_body(x_ref, w_ref, deg_ref, h2_ref):
    deg = deg_ref[0] + deg_ref[1] + 1.0
    dis = lax.rsqrt(deg)[:, None]
    h = jnp.dot(x_ref[...], w_ref[...], preferred_element_type=jnp.float32)
    h2_ref[...] = h * dis


def _mm_call(x_p, W, deg2):
    return pl.pallas_call(
        _mm_body,
        grid=(N_PAD // 1024,),
        in_specs=[
            pl.BlockSpec((1024, D), lambda i: (i, 0)),
            pl.BlockSpec((D, D), lambda i: (0, 0)),
            pl.BlockSpec((2, 1024), lambda i: (0, i)),
        ],
        out_specs=pl.BlockSpec((1024, D), lambda i: (i, 0)),
        out_shape=jax.ShapeDtypeStruct((N_PAD, D), jnp.float32),
    )(x_p, W, deg2)


CH_F = 290
CH_S = (E_PAD // CHUNK) // 16 - CH_F


def _agg_body(h2_hbm, edge_hbm, part_hbm, idxv, gbuf, acc, isem, gsem):
    c = lax.axis_index("c")
    s = lax.axis_index("s")
    nch = jnp.where(c == 0, CH_F, CH_S)
    base = jnp.where(c == 0, s * CH_F, 16 * CH_F + s * CH_S)
    pltpu.sync_copy(h2_hbm.at[pl.ds(s * ROWS_PER_TILE, ROWS_PER_TILE)],
                    acc.at[pl.ds(s * ROWS_PER_TILE, ROWS_PER_TILE)])
    plsc.subcore_barrier()

    for p in range(12):
        pltpu.async_copy(edge_hbm.at[base + p], idxv.at[p], isem)
    for p in range(4):
        pltpu.make_async_copy(edge_hbm.at[base + p], idxv.at[p], isem).wait()
        pltpu.async_copy(h2_hbm.at[idxv.at[p, 0]], gbuf.at[p], gsem)

    def body(j, carry):
        b = lax.rem(j, 4)
        sl = lax.rem(j, 16)
        pltpu.make_async_copy(h2_hbm.at[idxv.at[sl, 0]], gbuf.at[b], gsem).wait()
        pltpu.sync_copy(gbuf.at[b], acc.at[idxv.at[sl, 1]], add=True)

        @pl.when(j + 4 < nch)
        def _():
            sl4 = lax.rem(j + 4, 16)
            pltpu.make_async_copy(edge_hbm.at[base + j + 4], idxv.at[sl4], isem).wait()
            pltpu.async_copy(h2_hbm.at[idxv.at[sl4, 0]], gbuf.at[b], gsem)

        @pl.when(j + 12 < nch)
        def _():
            sl12 = lax.rem(j + 12, 16)
            pltpu.async_copy(edge_hbm.at[base + j + 12], idxv.at[sl12], isem)

        return carry

    lax.fori_loop(0, nch, body, 0)
    plsc.subcore_barrier()
    pltpu.sync_copy(acc.at[pl.ds(s * ROWS_PER_TILE, ROWS_PER_TILE)],
                    part_hbm.at[c, pl.ds(s * ROWS_PER_TILE, ROWS_PER_TILE)])


_agg_call = functools.partial(
    pl.kernel,
    out_type=jax.ShapeDtypeStruct((2, N_PAD, D), jnp.float32),
    mesh=_mesh,
    scratch_types=[
        pltpu.VMEM((16, 2, CHUNK), jnp.int32),
        pltpu.VMEM((4, CHUNK, D), jnp.float32),
        pltpu.VMEM_SHARED((N_PAD, D), jnp.float32),
        pltpu.SemaphoreType.DMA,
        pltpu.SemaphoreType.DMA,
    ],
)(_agg_body)


def _fin_body(part_ref, h2_ref, deg_ref, b_ref, out_ref):
    deg = deg_ref[0] + deg_ref[1] + 1.0
    dis = lax.rsqrt(deg)[:, None]
    agg = part_ref[0] + part_ref[1] - h2_ref[...]
    out_ref[...] = agg * dis + b_ref[...]


def _fin_call(part, h2, deg2, b2):
    return pl.pallas_call(
        _fin_body,
        grid=(N_PAD // 1024,),
        in_specs=[
            pl.BlockSpec((2, 1024, D), lambda i: (0, i, 0)),
            pl.BlockSpec((1024, D), lambda i: (i, 0)),
            pl.BlockSpec((2, 1024), lambda i: (0, i)),
            pl.BlockSpec((1, D), lambda i: (0, 0)),
        ],
        out_specs=pl.BlockSpec((1024, D), lambda i: (i, 0)),
        out_shape=jax.ShapeDtypeStruct((N_NODES, D), jnp.float32),
    )(part, h2, deg2, b2)


def kernel(x, edge_index, W, b):
    src = edge_index[0].astype(jnp.int32)
    dst = edge_index[1].astype(jnp.int32)
    pad = E_PAD - src.shape[0]
    src_p = jnp.concatenate([src, jnp.zeros((pad,), jnp.int32)])
    dst_p = jnp.concatenate([dst, jnp.full((pad,), N_NODES, jnp.int32)])
    edge3 = jnp.stack(
        [src_p.reshape(E_PAD // CHUNK, CHUNK), dst_p.reshape(E_PAD // CHUNK, CHUNK)],
        axis=1)
    x_p = jnp.concatenate([x, jnp.zeros((N_PAD - N_NODES, D), x.dtype)])

    deg2, _ = _deg_call(dst_p)
    h2 = _mm_call(x_p, W, deg2)
    part = _agg_call(h2, edge3)
    return _fin_call(part, h2, deg2, b.reshape(1, D))
